# baseline (speedup 1.0000x reference)
import functools

import jax
import jax.numpy as jnp
from jax import lax
from jax.experimental import pallas as pl
from jax.experimental.pallas import tpu as pltpu
from jax.experimental.pallas import tpu_sc as plsc

_NC = 2
_NS = 16
_NW = _NC * _NS

_BV = 2048


def _sc_gather(table_pairs, idx_half):
    B = idx_half.shape[0]
    D2 = table_pairs.shape[1]
    b_per_w = B // _NW
    mesh = plsc.VectorSubcoreMesh(core_axis_name="c", subcore_axis_name="s")

    @functools.partial(
        pl.kernel,
        mesh=mesh,
        out_type=jax.ShapeDtypeStruct((B, D2), table_pairs.dtype),
        scratch_types=[
            pltpu.VMEM((b_per_w,), jnp.int32),
            pltpu.VMEM((b_per_w, D2), table_pairs.dtype),
            pltpu.SemaphoreType.DMA,
        ],
    )
    def gather_kernel(tp_hbm, idx_hbm, out_hbm, idx_v, rows_v, sem):
        wid = lax.axis_index("s") * _NC + lax.axis_index("c")
        base = wid * b_per_w
        pltpu.sync_copy(idx_hbm.at[pl.ds(base, b_per_w)], idx_v)
        pltpu.async_copy(tp_hbm.at[idx_v], rows_v, sem).wait()
        pltpu.sync_copy(rows_v, out_hbm.at[pl.ds(base, b_per_w)])

    return gather_kernel(table_pairs, idx_half)


def _lm_head_kernel(xw_ref, par_ref, wt_ref, b_ref, o_ref):
    D = wt_ref.shape[0]
    xw = xw_ref[...]
    x = jnp.where(par_ref[...] != 0, xw[:, D:], xw[:, :D]).astype(jnp.bfloat16)
    wt = wt_ref[...].astype(jnp.bfloat16)
    acc = lax.dot_general(
        wt, x, (((0,), (1,)), ((), ())), preferred_element_type=jnp.float32
    )
    ones = jnp.ones((1, x.shape[0]), dtype=jnp.bfloat16)
    bcast = lax.dot_general(
        b_ref[...].astype(jnp.bfloat16),
        ones,
        (((0,), (0,)), ((), ())),
        preferred_element_type=jnp.float32,
    )
    o_ref[...] = acc + bcast


def _lm_head_t(x_wide, parity, wt, brow):
    B = x_wide.shape[0]
    D2 = x_wide.shape[1]
    D, V = wt.shape
    return pl.pallas_call(
        _lm_head_kernel,
        grid=(pl.cdiv(V, _BV),),
        in_specs=[
            pl.BlockSpec((B, D2), lambda i: (0, 0)),
            pl.BlockSpec((B, 1), lambda i: (0, 0)),
            pl.BlockSpec((D, _BV), lambda i: (0, i)),
            pl.BlockSpec((1, _BV), lambda i: (0, i)),
        ],
        out_specs=pl.BlockSpec((_BV, B), lambda i: (i, 0)),
        out_shape=jax.ShapeDtypeStruct((V, B), jnp.float32),
    )(x_wide, parity, wt, brow)


def kernel(input_ids, emb_table, lm_head_w, lm_head_b):
    ids = input_ids.astype(jnp.int32)
    V, D = emb_table.shape
    table_pairs = emb_table.reshape(V // 2, 2 * D)
    x_wide = _sc_gather(table_pairs, ids >> 1)
    parity = (ids & 1).reshape(-1, 1)
    logits_t = _lm_head_t(
        x_wide, parity, lm_head_w.T, lm_head_b.reshape(1, -1)
    )
    return logits_t.T


# speedup vs baseline: 1.1156x; 1.1156x over previous
import functools

import jax
import jax.numpy as jnp
from jax import lax
from jax.experimental import pallas as pl
from jax.experimental.pallas import tpu as pltpu
from jax.experimental.pallas import tpu_sc as plsc

_NC = 2

_BT = 8192
_BV = 4096


def _transpose_kernel(xt_ref, o_ref):
    o_ref[...] = xt_ref[...].T


def _rowize_table(embt):
    D, V = embt.shape
    return pl.pallas_call(
        _transpose_kernel,
        grid=(pl.cdiv(V, _BT),),
        in_specs=[pl.BlockSpec((D, _BT), lambda i: (0, i))],
        out_specs=pl.BlockSpec((_BT, D), lambda i: (i, 0)),
        out_shape=jax.ShapeDtypeStruct((V, D), embt.dtype),
    )(embt)


def _sc_gather(table, idx):
    B = idx.shape[0]
    D = table.shape[1]
    b_per_w = B // _NC
    mesh = plsc.ScalarSubcoreMesh(axis_name="c", num_cores=_NC)

    @functools.partial(
        pl.kernel,
        mesh=mesh,
        out_type=jax.ShapeDtypeStruct((B, D), table.dtype),
        scratch_types=[
            pltpu.SMEM((b_per_w,), jnp.int32),
            pltpu.SemaphoreType.DMA,
            pltpu.SemaphoreType.DMA,
        ],
    )
    def gather_kernel(table_hbm, idx_hbm, out_hbm, idx_s, isem, sem):
        cid = lax.axis_index("c")
        base = cid * b_per_w
        pltpu.async_copy(idx_hbm.at[pl.ds(base, b_per_w)], idx_s, isem).wait()

        @pl.loop(0, b_per_w)
        def _(j):
            pltpu.make_async_copy(
                table_hbm.at[idx_s[j]], out_hbm.at[base + j], sem
            ).start()

        pltpu.make_async_copy(
            table_hbm.at[pl.ds(0, b_per_w)],
            out_hbm.at[pl.ds(base, b_per_w)],
            sem,
        ).wait()

    return gather_kernel(table, idx)


def _lm_head_kernel(x_ref, wt_ref, b_ref, o_ref):
    x = x_ref[...].astype(jnp.bfloat16)
    wt = wt_ref[...].astype(jnp.bfloat16)
    acc = lax.dot_general(
        wt, x, (((0,), (1,)), ((), ())), preferred_element_type=jnp.float32
    )
    ones = jnp.ones((1, x.shape[0]), dtype=jnp.bfloat16)
    bcast = lax.dot_general(
        b_ref[...].astype(jnp.bfloat16),
        ones,
        (((0,), (0,)), ((), ())),
        preferred_element_type=jnp.float32,
    )
    o_ref[...] = acc + bcast


def _lm_head_t(x, wt, brow):
    B, D = x.shape
    V = wt.shape[1]
    return pl.pallas_call(
        _lm_head_kernel,
        grid=(pl.cdiv(V, _BV),),
        in_specs=[
            pl.BlockSpec((B, D), lambda i: (0, 0)),
            pl.BlockSpec((D, _BV), lambda i: (0, i)),
            pl.BlockSpec((1, _BV), lambda i: (0, i)),
        ],
        out_specs=pl.BlockSpec((_BV, B), lambda i: (i, 0)),
        out_shape=jax.ShapeDtypeStruct((V, B), jnp.float32),
    )(x, wt, brow)


def kernel(input_ids, emb_table, lm_head_w, lm_head_b):
    ids = input_ids.astype(jnp.int32)
    table_rows = _rowize_table(emb_table.T)
    x = _sc_gather(table_rows, ids)
    logits_t = _lm_head_t(x, lm_head_w.T, lm_head_b.reshape(1, -1))
    return logits_t.T
